# Initial kernel scaffold; baseline (speedup 1.0000x reference)
import jax
import jax.numpy as jnp
from jax.experimental import pallas as pl


def kernel(x, index):
    raise NotImplementedError("write your pallas kernel here")


# speedup vs baseline: 4.4474x; 4.4474x over previous
import functools

import jax
import jax.numpy as jnp
from jax import lax
from jax.experimental import pallas as pl
from jax.experimental.pallas import tpu as pltpu
from jax.experimental.pallas import tpu_sc as plsc

NC = 2
NS = 16
NW = NC * NS
LANES = 16
CH = 128


def _sc_partials_kernel(N, DH, NCH, ACC_ROWS, RPW):
    mesh = plsc.VectorSubcoreMesh(core_axis_name="c", subcore_axis_name="s")

    @functools.partial(
        pl.kernel,
        mesh=mesh,
        out_type=jax.ShapeDtypeStruct((NC, 2, ACC_ROWS, DH), jnp.float32),
        scratch_types=[
            pltpu.VMEM((NCH, CH), jnp.int32),
            pltpu.VMEM((NCH, CH), jnp.int32),
            pltpu.VMEM((CH, DH), jnp.float32),
            pltpu.VMEM((CH, DH), jnp.float32),
            pltpu.VMEM_SHARED((ACC_ROWS, DH), jnp.float32),
        ],
        compiler_params=pltpu.CompilerParams(use_tc_tiling_on_sc=False),
    )
    def k(x0_hbm, x1_hbm, idx0_hbm, idx1_hbm, out_hbm, idx0_v, idx1_v,
          rows_v, zeros_v, acc_sh):
        c = lax.axis_index("c")
        s = lax.axis_index("s")
        w = c * NS + s

        pltpu.sync_copy(idx0_hbm.at[w], idx0_v)
        pltpu.sync_copy(idx1_hbm.at[w], idx1_v)

        zvec = jnp.zeros((LANES,), jnp.float32)

        def zrow(i, _):
            for j in range(DH // LANES):
                zeros_v[i, pl.ds(j * LANES, LANES)] = zvec
            return 0

        lax.fori_loop(0, CH, zrow, 0)

        base = s * RPW
        full = RPW // CH
        rem = RPW - full * CH

        for h, xh in ((0, x0_hbm), (1, x1_hbm)):
            for t in range(full):
                pltpu.sync_copy(zeros_v, acc_sh.at[pl.ds(base + t * CH, CH)])
            if rem:
                pltpu.sync_copy(zeros_v.at[pl.ds(0, rem)],
                                acc_sh.at[pl.ds(base + full * CH, rem)])
            plsc.subcore_barrier()

            def body(j, _):
                pltpu.sync_copy(xh.at[idx0_v.at[j]], rows_v)
                pltpu.sync_copy(rows_v, acc_sh.at[idx1_v.at[j]], add=True)
                return 0

            lax.fori_loop(0, NCH, body, 0)
            plsc.subcore_barrier()

            for t in range(full):
                pltpu.sync_copy(acc_sh.at[pl.ds(base + t * CH, CH)], rows_v)
                pltpu.sync_copy(rows_v,
                                out_hbm.at[c, h, pl.ds(base + t * CH, CH)])
            if rem:
                pltpu.sync_copy(acc_sh.at[pl.ds(base + full * CH, rem)],
                                rows_v.at[pl.ds(0, rem)])
                pltpu.sync_copy(rows_v.at[pl.ds(0, rem)],
                                out_hbm.at[c, h, pl.ds(base + full * CH, rem)])

    return k


def _add_body(a_ref, b_ref, o_ref):
    o_ref[...] = jnp.concatenate(
        [a_ref[0, 0] + b_ref[0, 0], a_ref[0, 1] + b_ref[0, 1]], axis=1)


def kernel(x, index):
    N, D = x.shape
    DH = D // 2
    E = index.shape[1]
    assert E % NW == 0 and D % 2 == 0
    EW = E // NW
    NCH = -(-EW // CH)
    EP = NCH * CH
    ACC_ROWS = -(-(N + 1) // (NS * 8)) * (NS * 8)
    RPW = ACC_ROWS // NS

    idx = index.astype(jnp.int32)
    i0 = jnp.pad(idx[0].reshape(NW, EW), ((0, 0), (0, EP - EW)))
    i1 = jnp.pad(idx[1].reshape(NW, EW), ((0, 0), (0, EP - EW)),
                 constant_values=N)
    i0 = i0.reshape(NW, NCH, CH)
    i1 = i1.reshape(NW, NCH, CH)

    x0 = x[:, :DH]
    x1 = x[:, DH:]
    partials = _sc_partials_kernel(N, DH, NCH, ACC_ROWS, RPW)(x0, x1, i0, i1)

    BR = 2000
    out = pl.pallas_call(
        _add_body,
        out_shape=jax.ShapeDtypeStruct((N, D), jnp.float32),
        grid=(N // BR,),
        in_specs=[pl.BlockSpec((1, 2, BR, DH), lambda i: (0, 0, i, 0)),
                  pl.BlockSpec((1, 2, BR, DH), lambda i: (1, 0, i, 0))],
        out_specs=pl.BlockSpec((BR, D), lambda i: (i, 0)),
    )(partials, partials)
    return out
